# baseline (device time: 25311 ns/iter reference)
import functools
import math

import jax
import jax.numpy as jnp
from jax import lax
from jax.experimental import pallas as pl
from jax.experimental.pallas import tpu as pltpu

N_DEV = 4
BLK = 64


def kernel(x, Wq, K_ext, V_ext, Wo):
    B, Sq, Dm = x.shape
    _, Skv, Hq, Dh = K_ext.shape
    HD = Hq * Dh
    Dout = Wo.shape[1]
    J = Sq // BLK

    K2 = K_ext.reshape(B, Skv, HD)
    V2 = V_ext.reshape(B, Skv, HD)

    def body(x_ref, wq_ref, k_ref, v_ref, wo_ref, out_ref,
             kvsel, sends, recvs):
        my = lax.axis_index("i")
        right = (my + 1) % N_DEV
        left = (my + 3) % N_DEV

        barrier = pltpu.get_barrier_semaphore()
        for p in (left, right):
            pl.semaphore_signal(barrier, inc=1, device_id=(p,),
                                device_id_type=pl.DeviceIdType.MESH)
        pl.semaphore_wait(barrier, 2)

        kvsel[0, :, :, 0] = k_ref[...].astype(jnp.bfloat16).reshape(B, J, BLK, HD)
        kvsel[1, :, :, 0] = v_ref[...].astype(jnp.bfloat16).reshape(B, J, BLK, HD)

        def copy(src_slice, dst_slice, sem_i, dst_dev):
            return pltpu.make_async_remote_copy(
                src_ref=src_slice, dst_ref=dst_slice,
                send_sem=sends.at[sem_i], recv_sem=recvs.at[sem_i],
                device_id=(dst_dev,), device_id_type=pl.DeviceIdType.MESH)

        d1 = copy(kvsel.at[:, :, :, 0], kvsel.at[:, :, :, 1], 1, left)
        d2 = copy(kvsel.at[:, :, :, 0], kvsel.at[:, :, :, 2], 2, right)
        d1.start()
        d2.start()

        wq = wq_ref[...].astype(jnp.bfloat16)
        wo = wo_ref[...].astype(jnp.bfloat16)
        scale = 0.125 * math.log2(math.e)
        q = [(jnp.dot(x_ref[b].astype(jnp.bfloat16), wq,
                      preferred_element_type=jnp.float32) * scale
              ).astype(jnp.bfloat16) for b in range(B)]

        acc = {}

        def process(lo, hi):
            n = hi - lo
            for b in range(B):
                for j in range(J):
                    kk = kvsel[0, b, j, lo:hi].reshape(n * BLK, HD)
                    vv = kvsel[1, b, j, lo:hi].reshape(n * BLK, HD)
                    q_blk = q[b][j * BLK:(j + 1) * BLK, :]
                    for hh in range(Hq):
                        cs = slice(hh * Dh, (hh + 1) * Dh)
                        s = lax.dot_general(
                            q_blk[:, cs], kk[:, cs],
                            (((1,), (1,)), ((), ())),
                            preferred_element_type=jnp.float32)
                        e = jnp.exp2(s)
                        l = jnp.sum(e, axis=-1, keepdims=True)
                        c = jnp.dot(e.astype(jnp.bfloat16), vv[:, cs],
                                    preferred_element_type=jnp.float32)
                        if (b, j, hh) in acc:
                            a = acc[(b, j, hh)]
                            acc[(b, j, hh)] = [a[0] + c, a[1] + l]
                        else:
                            acc[(b, j, hh)] = [c, l]

        process(0, 1)

        d1.wait_recv()
        r1 = copy(kvsel.at[:, 0, :, 1], kvsel.at[:, 0, :, 3], 3, left)
        r1.start()
        d2.wait_recv()
        r2 = copy(kvsel.at[:, 1, :, 2], kvsel.at[:, 1, :, 3], 4, right)
        r2.start()

        process(1, 3)

        r1.wait_recv()
        r2.wait_recv()
        process(3, 4)

        for b in range(B):
            ctx_rows = []
            for j in range(J):
                ctx_rows.append(jnp.concatenate(
                    [(acc[(b, j, hh)][0] * (1.0 / acc[(b, j, hh)][1])
                      ).astype(jnp.bfloat16) for hh in range(Hq)], axis=1))
            ctx_b = jnp.concatenate(ctx_rows, axis=0)
            out_ref[b] = jnp.dot(ctx_b, wo, preferred_element_type=jnp.float32)

        for r in (d1, d2, r1, r2):
            r.wait_send()

        @functools.partial(pl.run_scoped,
                           second_barrier=pltpu.SemaphoreType.REGULAR)
        def _(second_barrier):
            for p in (left, right):
                pl.semaphore_signal(second_barrier, inc=1, device_id=(p,),
                                    device_id_type=pl.DeviceIdType.MESH)
            pl.semaphore_wait(second_barrier, 2)

    return pl.pallas_call(
        body,
        out_shape=jax.ShapeDtypeStruct((B, Sq, Dout), jnp.float32),
        in_specs=[pl.BlockSpec(memory_space=pltpu.VMEM)] * 5,
        out_specs=pl.BlockSpec(memory_space=pltpu.VMEM),
        scratch_shapes=[
            pltpu.VMEM((2, B, J, N_DEV, BLK, HD), jnp.bfloat16),
            pltpu.SemaphoreType.DMA((5,)),
            pltpu.SemaphoreType.DMA((5,)),
        ],
        compiler_params=pltpu.CompilerParams(collective_id=0),
    )(x, Wq, K2, V2, Wo)
